# Initial kernel scaffold; baseline (speedup 1.0000x reference)
import jax
import jax.numpy as jnp
from jax.experimental import pallas as pl


def kernel(surface_x, graph_x, edge_index_gs, edge_weight_gs, edge_index_sg, edge_weight_sg, W_s_pre, b_s_pre, W_g_pre, b_g_pre, W_gs, W_sg, W_s_post, b_s_post, W_g_post, b_g_post):
    raise NotImplementedError("write your pallas kernel here")


# speedup vs baseline: 2.2115x; 2.2115x over previous
import functools

import jax
import jax.numpy as jnp
from jax import lax
from jax.experimental import pallas as pl
from jax.experimental.pallas import tpu as pltpu
from jax.experimental.pallas import tpu_sc as plsc

D = 128
LANES = 16
SEG = D // LANES
NUM_TILES = 32
C = 128
ZR = 128


def _mm_bias_body(x_ref, w_ref, b_ref, o_ref):
    o_ref[...] = (
        jnp.dot(x_ref[...], w_ref[...], preferred_element_type=jnp.float32)
        + b_ref[...]
    )


def _pre_graph_body(gx_ref, wg_ref, bg_ref, wgs_ref, xg_ref, xgt_ref):
    xg = (
        jnp.dot(gx_ref[...], wg_ref[...], preferred_element_type=jnp.float32)
        + bg_ref[...]
    )
    xg_ref[...] = xg
    xgt_ref[...] = jnp.dot(xg, wgs_ref[...], preferred_element_type=jnp.float32)


def _post_surface_body(xs_ref, g0_ref, g1_ref, w_ref, b_ref, o_ref):
    agg = g0_ref[...] + g1_ref[...]
    acc = (
        jnp.dot(xs_ref[...], w_ref[0:D, :], preferred_element_type=jnp.float32)
        + jnp.dot(agg, w_ref[D:2 * D, :], preferred_element_type=jnp.float32)
        + b_ref[...]
    )
    o_ref[...] = jnp.maximum(acc, 0.0)


def _post_graph_body(xg_ref, s0_ref, s1_ref, wsg_ref, w_ref, b_ref, o_ref):
    agg = s0_ref[...] + s1_ref[...]
    xg_out = jnp.dot(agg, wsg_ref[...], preferred_element_type=jnp.float32)
    acc = (
        jnp.dot(xg_ref[...], w_ref[0:D, :], preferred_element_type=jnp.float32)
        + jnp.dot(xg_out, w_ref[D:2 * D, :], preferred_element_type=jnp.float32)
        + b_ref[...]
    )
    o_ref[...] = jnp.maximum(acc, 0.0)


def _make_sc_mp(Nsp, Ngp, chunks):
    rows_gs = Nsp // 16
    rows_sg = Ngp // 16
    mesh = plsc.VectorSubcoreMesh(core_axis_name="c", subcore_axis_name="s")

    @functools.partial(
        pl.kernel,
        mesh=mesh,
        out_type=[
            jax.ShapeDtypeStruct((2, Nsp, D), jnp.float32),
            jax.ShapeDtypeStruct((2, Ngp, D), jnp.float32),
        ],
        scratch_types=[
            pltpu.VMEM((chunks, C), jnp.int32),
            pltpu.VMEM((chunks, C), jnp.int32),
            pltpu.VMEM((chunks, C), jnp.float32),
            pltpu.VMEM((C, D), jnp.float32),
            pltpu.VMEM((ZR, D), jnp.float32),
            pltpu.VMEM_SHARED((Nsp, D), jnp.float32),
            pltpu.VMEM_SHARED((Ngp, D), jnp.float32),
            pltpu.SemaphoreType.DMA,
        ],
    )
    def sc_mp(tbl_gs, tbl_sg, src_gs, dst_gs, w_gs, src_sg, dst_sg, w_sg,
              out_gs, out_sg,
              src_v, dst_v, w_v, rows_v, zb_v, acc_gs, acc_sg, sem):
        cid = lax.axis_index("c")
        sid = lax.axis_index("s")
        wid = cid * 16 + sid

        zeros16 = jnp.zeros((LANES,), jnp.float32)

        def zrow(i, carry):
            for j in range(SEG):
                zb_v[i, pl.ds(j * LANES, LANES)] = zeros16
            return carry

        lax.fori_loop(0, ZR, zrow, 0)

        for r in range(rows_gs // ZR):
            pltpu.sync_copy(zb_v, acc_gs.at[pl.ds(sid * rows_gs + r * ZR, ZR)])
        for r in range(rows_sg // ZR):
            pltpu.sync_copy(zb_v, acc_sg.at[pl.ds(sid * rows_sg + r * ZR, ZR)])
        plsc.subcore_barrier()

        def run_direction(tbl, src_h, dst_h, w_h, acc):
            pltpu.sync_copy(src_h.at[wid], src_v)
            pltpu.sync_copy(dst_h.at[wid], dst_v)
            pltpu.sync_copy(w_h.at[wid], w_v)

            def chunk_body(ci, carry):
                pltpu.async_copy(tbl.at[src_v.at[ci]], rows_v, sem).wait()

                def group_body(g, carry2):
                    wvec = w_v[ci, pl.ds(g * LANES, LANES)]
                    for u in range(LANES):
                        e = g * LANES + u
                        wv = jnp.full((LANES,), wvec[u], jnp.float32)
                        for j in range(SEG):
                            sl = pl.ds(j * LANES, LANES)
                            rows_v[e, sl] = rows_v[e, sl] * wv
                    return carry2

                lax.fori_loop(0, C // LANES, group_body, 0)

                pltpu.sync_copy(rows_v, acc.at[dst_v.at[ci]], add=True)
                return carry

            lax.fori_loop(0, chunks, chunk_body, 0)

        run_direction(tbl_gs, src_gs, dst_gs, w_gs, acc_gs)
        run_direction(tbl_sg, src_sg, dst_sg, w_sg, acc_sg)
        plsc.subcore_barrier()

        for r in range(rows_gs // ZR):
            row0 = sid * rows_gs + r * ZR
            pltpu.sync_copy(acc_gs.at[pl.ds(row0, ZR)], zb_v)
            pltpu.sync_copy(zb_v, out_gs.at[cid, pl.ds(row0, ZR)])
        for r in range(rows_sg // ZR):
            row0 = sid * rows_sg + r * ZR
            pltpu.sync_copy(acc_sg.at[pl.ds(row0, ZR)], zb_v)
            pltpu.sync_copy(zb_v, out_sg.at[cid, pl.ds(row0, ZR)])

    return sc_mp


def _pad_edges(src, dst, w, ep):
    e = src.shape[0]
    pad = ep - e
    if pad:
        src = jnp.concatenate([src, jnp.zeros((pad,), jnp.int32)])
        dst = jnp.concatenate([dst, jnp.zeros((pad,), jnp.int32)])
        w = jnp.concatenate([w, jnp.zeros((pad,), jnp.float32)])
    chunks = ep // (NUM_TILES * C)
    return (src.reshape(NUM_TILES, chunks, C),
            dst.reshape(NUM_TILES, chunks, C),
            w.reshape(NUM_TILES, chunks, C))


def kernel(surface_x, graph_x, edge_index_gs, edge_weight_gs, edge_index_sg,
           edge_weight_sg, W_s_pre, b_s_pre, W_g_pre, b_g_pre, W_gs, W_sg,
           W_s_post, b_s_post, W_g_post, b_g_post):
    Ns = surface_x.shape[0]
    Ng = graph_x.shape[0]
    E = edge_weight_gs.shape[0]
    SBLK = Ns // 8

    bs = b_s_pre.reshape(1, D)
    bg = b_g_pre.reshape(1, D)
    bsp = b_s_post.reshape(1, D)
    bgp = b_g_post.reshape(1, D)

    xs = pl.pallas_call(
        _mm_bias_body,
        grid=(8,),
        in_specs=[
            pl.BlockSpec((SBLK, D), lambda i: (i, 0)),
            pl.BlockSpec((D, D), lambda i: (0, 0)),
            pl.BlockSpec((1, D), lambda i: (0, 0)),
        ],
        out_specs=pl.BlockSpec((SBLK, D), lambda i: (i, 0)),
        out_shape=jax.ShapeDtypeStruct((Ns, D), jnp.float32),
    )(surface_x, W_s_pre, bs)

    xg, xg_t = pl.pallas_call(
        _pre_graph_body,
        out_shape=[
            jax.ShapeDtypeStruct((Ng, D), jnp.float32),
            jax.ShapeDtypeStruct((Ng, D), jnp.float32),
        ],
    )(graph_x, W_g_pre, bg, W_gs)

    ep = -(-E // (NUM_TILES * C)) * (NUM_TILES * C)
    chunks = ep // (NUM_TILES * C)
    src_gs, dst_gs, w_gs = _pad_edges(
        edge_index_gs[0] - Ns, edge_index_gs[1], edge_weight_gs, ep)
    src_sg, dst_sg, w_sg = _pad_edges(
        edge_index_sg[0], edge_index_sg[1] - Ns, edge_weight_sg, ep)

    nsp = -(-Ns // (16 * ZR)) * (16 * ZR)
    ngp = -(-Ng // (16 * ZR)) * (16 * ZR)
    out_gs, out_sg = _make_sc_mp(nsp, ngp, chunks)(
        xg_t, xs, src_gs, dst_gs, w_gs, src_sg, dst_sg, w_sg)
    out_gs = out_gs[:, :Ns]
    out_sg = out_sg[:, :Ng]

    xs_new = pl.pallas_call(
        _post_surface_body,
        grid=(8,),
        in_specs=[
            pl.BlockSpec((SBLK, D), lambda i: (i, 0)),
            pl.BlockSpec((SBLK, D), lambda i: (i, 0)),
            pl.BlockSpec((SBLK, D), lambda i: (i, 0)),
            pl.BlockSpec((2 * D, D), lambda i: (0, 0)),
            pl.BlockSpec((1, D), lambda i: (0, 0)),
        ],
        out_specs=pl.BlockSpec((SBLK, D), lambda i: (i, 0)),
        out_shape=jax.ShapeDtypeStruct((Ns, D), jnp.float32),
    )(xs, out_gs[0], out_gs[1], W_s_post, bsp)

    xg_new = pl.pallas_call(
        _post_graph_body,
        out_shape=jax.ShapeDtypeStruct((Ng, D), jnp.float32),
    )(xg, out_sg[0], out_sg[1], W_sg, W_g_post, bgp)

    return (xs_new, xg_new)


# speedup vs baseline: 2.5209x; 1.1399x over previous
import functools

import jax
import jax.numpy as jnp
from jax import lax
from jax.experimental import pallas as pl
from jax.experimental.pallas import tpu as pltpu
from jax.experimental.pallas import tpu_sc as plsc

D = 128
LANES = 16
SEG = D // LANES
NUM_TILES = 32
C = 128
ZR = 128


def _mm_bias_body(x_ref, w_ref, b_ref, o_ref):
    o_ref[...] = (
        jnp.dot(x_ref[...], w_ref[...], preferred_element_type=jnp.float32)
        + b_ref[...]
    )


def _pre_graph_body(gx_ref, wg_ref, bg_ref, wgs_ref, xg_ref, xgt_ref):
    xg = (
        jnp.dot(gx_ref[...], wg_ref[...], preferred_element_type=jnp.float32)
        + bg_ref[...]
    )
    xg_ref[...] = xg
    xgt_ref[...] = jnp.dot(xg, wgs_ref[...], preferred_element_type=jnp.float32)


def _post_surface_body(xs_ref, g0_ref, g1_ref, w_ref, b_ref, o_ref):
    agg = g0_ref[...] + g1_ref[...]
    acc = (
        jnp.dot(xs_ref[...], w_ref[0:D, :], preferred_element_type=jnp.float32)
        + jnp.dot(agg, w_ref[D:2 * D, :], preferred_element_type=jnp.float32)
        + b_ref[...]
    )
    o_ref[...] = jnp.maximum(acc, 0.0)


def _post_graph_body(xg_ref, s0_ref, s1_ref, wsg_ref, w_ref, b_ref, o_ref):
    agg = s0_ref[...] + s1_ref[...]
    xg_out = jnp.dot(agg, wsg_ref[...], preferred_element_type=jnp.float32)
    acc = (
        jnp.dot(xg_ref[...], w_ref[0:D, :], preferred_element_type=jnp.float32)
        + jnp.dot(xg_out, w_ref[D:2 * D, :], preferred_element_type=jnp.float32)
        + b_ref[...]
    )
    o_ref[...] = jnp.maximum(acc, 0.0)


def _make_sc_mp(Nsp, Ngp, chunks):
    rows_gs = Nsp // 16
    rows_sg = Ngp // 16
    mesh = plsc.VectorSubcoreMesh(core_axis_name="c", subcore_axis_name="s")

    @functools.partial(
        pl.kernel,
        mesh=mesh,
        out_type=[
            jax.ShapeDtypeStruct((2, Nsp, D), jnp.float32),
            jax.ShapeDtypeStruct((2, Ngp, D), jnp.float32),
        ],
        scratch_types=[
            pltpu.VMEM((chunks, C), jnp.int32),
            pltpu.VMEM((chunks, C), jnp.int32),
            pltpu.VMEM((chunks, C), jnp.float32),
            pltpu.VMEM((C, D), jnp.float32),
            pltpu.VMEM((C, D), jnp.float32),
            pltpu.VMEM_SHARED((Nsp, D), jnp.float32),
            pltpu.VMEM_SHARED((Ngp, D), jnp.float32),
            pltpu.SemaphoreType.DMA,
            pltpu.SemaphoreType.DMA,
            pltpu.SemaphoreType.DMA,
            pltpu.SemaphoreType.DMA,
        ],
    )
    def sc_mp(tbl_gs, tbl_sg, src_gs, dst_gs, w_gs, src_sg, dst_sg, w_sg,
              out_gs, out_sg,
              src_v, dst_v, w_v, rows0, rows1,
              acc_gs, acc_sg, g0, g1, s0, s1):
        rows = (rows0, rows1)
        gsems = (g0, g1)
        ssems = (s0, s1)
        zb_v = rows0
        cid = lax.axis_index("c")
        sid = lax.axis_index("s")
        wid = cid * 16 + sid

        zeros16 = jnp.zeros((LANES,), jnp.float32)

        def zrow(i, carry):
            for j in range(SEG):
                zb_v[i, pl.ds(j * LANES, LANES)] = zeros16
            return carry

        lax.fori_loop(0, ZR, zrow, 0)

        for r in range(rows_gs // ZR):
            pltpu.sync_copy(zb_v, acc_gs.at[pl.ds(sid * rows_gs + r * ZR, ZR)])
        for r in range(rows_sg // ZR):
            pltpu.sync_copy(zb_v, acc_sg.at[pl.ds(sid * rows_sg + r * ZR, ZR)])
        plsc.subcore_barrier()

        nbuf = len(rows)
        groups = chunks // nbuf

        def run_direction(tbl, src_h, dst_h, w_h, acc):
            pltpu.sync_copy(src_h.at[wid], src_v)
            pltpu.sync_copy(dst_h.at[wid], dst_v)
            pltpu.sync_copy(w_h.at[wid], w_v)

            def scale(buf, ci):
                def group_body(g, carry2):
                    wvec = w_v[ci, pl.ds(g * LANES, LANES)]
                    for u in range(LANES):
                        e = g * LANES + u
                        wv = jnp.full((LANES,), wvec[u], jnp.float32)
                        for j in range(SEG):
                            sl = pl.ds(j * LANES, LANES)
                            buf[e, sl] = buf[e, sl] * wv
                    return carry2

                lax.fori_loop(0, C // LANES, group_body, 0)

            for b in range(nbuf):
                pltpu.async_copy(tbl.at[src_v.at[b]], rows[b], gsems[b])

            def pipe_body(g, carry):
                handles = []
                for b in range(nbuf):
                    ci = g * nbuf + b
                    pltpu.make_async_copy(
                        tbl.at[src_v.at[ci]], rows[b], gsems[b]).wait()
                    scale(rows[b], ci)
                    handles.append(pltpu.async_copy(
                        rows[b], acc.at[dst_v.at[ci]], ssems[b], add=True))
                for b in range(nbuf):
                    handles[b].wait()

                    @pl.when(g < groups - 1)
                    def _():
                        nci = (g + 1) * nbuf + b
                        pltpu.async_copy(tbl.at[src_v.at[nci]], rows[b],
                                         gsems[b])
                return carry

            lax.fori_loop(0, groups, pipe_body, 0)

        run_direction(tbl_gs, src_gs, dst_gs, w_gs, acc_gs)
        run_direction(tbl_sg, src_sg, dst_sg, w_sg, acc_sg)
        plsc.subcore_barrier()

        for r in range(rows_gs // ZR):
            row0 = sid * rows_gs + r * ZR
            pltpu.sync_copy(acc_gs.at[pl.ds(row0, ZR)], zb_v)
            pltpu.sync_copy(zb_v, out_gs.at[cid, pl.ds(row0, ZR)])
        for r in range(rows_sg // ZR):
            row0 = sid * rows_sg + r * ZR
            pltpu.sync_copy(acc_sg.at[pl.ds(row0, ZR)], zb_v)
            pltpu.sync_copy(zb_v, out_sg.at[cid, pl.ds(row0, ZR)])

    return sc_mp


def _pad_edges(src, dst, w, ep):
    e = src.shape[0]
    pad = ep - e
    if pad:
        src = jnp.concatenate([src, jnp.zeros((pad,), jnp.int32)])
        dst = jnp.concatenate([dst, jnp.zeros((pad,), jnp.int32)])
        w = jnp.concatenate([w, jnp.zeros((pad,), jnp.float32)])
    chunks = ep // (NUM_TILES * C)
    return (src.reshape(NUM_TILES, chunks, C),
            dst.reshape(NUM_TILES, chunks, C),
            w.reshape(NUM_TILES, chunks, C))


def kernel(surface_x, graph_x, edge_index_gs, edge_weight_gs, edge_index_sg,
           edge_weight_sg, W_s_pre, b_s_pre, W_g_pre, b_g_pre, W_gs, W_sg,
           W_s_post, b_s_post, W_g_post, b_g_post):
    Ns = surface_x.shape[0]
    Ng = graph_x.shape[0]
    E = edge_weight_gs.shape[0]
    SBLK = Ns // 8

    bs = b_s_pre.reshape(1, D)
    bg = b_g_pre.reshape(1, D)
    bsp = b_s_post.reshape(1, D)
    bgp = b_g_post.reshape(1, D)

    xs = pl.pallas_call(
        _mm_bias_body,
        grid=(8,),
        in_specs=[
            pl.BlockSpec((SBLK, D), lambda i: (i, 0)),
            pl.BlockSpec((D, D), lambda i: (0, 0)),
            pl.BlockSpec((1, D), lambda i: (0, 0)),
        ],
        out_specs=pl.BlockSpec((SBLK, D), lambda i: (i, 0)),
        out_shape=jax.ShapeDtypeStruct((Ns, D), jnp.float32),
    )(surface_x, W_s_pre, bs)

    xg, xg_t = pl.pallas_call(
        _pre_graph_body,
        out_shape=[
            jax.ShapeDtypeStruct((Ng, D), jnp.float32),
            jax.ShapeDtypeStruct((Ng, D), jnp.float32),
        ],
    )(graph_x, W_g_pre, bg, W_gs)

    ep = -(-E // (NUM_TILES * C * 2)) * (NUM_TILES * C * 2)
    chunks = ep // (NUM_TILES * C)
    src_gs, dst_gs, w_gs = _pad_edges(
        edge_index_gs[0] - Ns, edge_index_gs[1], edge_weight_gs, ep)
    src_sg, dst_sg, w_sg = _pad_edges(
        edge_index_sg[0], edge_index_sg[1] - Ns, edge_weight_sg, ep)

    nsp = -(-Ns // (16 * ZR)) * (16 * ZR)
    ngp = -(-Ng // (16 * ZR)) * (16 * ZR)
    out_gs, out_sg = _make_sc_mp(nsp, ngp, chunks)(
        xg_t, xs, src_gs, dst_gs, w_gs, src_sg, dst_sg, w_sg)
    out_gs = out_gs[:, :Ns]
    out_sg = out_sg[:, :Ng]

    xs_new = pl.pallas_call(
        _post_surface_body,
        grid=(8,),
        in_specs=[
            pl.BlockSpec((SBLK, D), lambda i: (i, 0)),
            pl.BlockSpec((SBLK, D), lambda i: (i, 0)),
            pl.BlockSpec((SBLK, D), lambda i: (i, 0)),
            pl.BlockSpec((2 * D, D), lambda i: (0, 0)),
            pl.BlockSpec((1, D), lambda i: (0, 0)),
        ],
        out_specs=pl.BlockSpec((SBLK, D), lambda i: (i, 0)),
        out_shape=jax.ShapeDtypeStruct((Ns, D), jnp.float32),
    )(xs, out_gs[0], out_gs[1], W_s_post, bsp)

    xg_new = pl.pallas_call(
        _post_graph_body,
        out_shape=jax.ShapeDtypeStruct((Ng, D), jnp.float32),
    )(xg, out_sg[0], out_sg[1], W_sg, W_g_post, bgp)

    return (xs_new, xg_new)
